# baseline (speedup 1.0000x reference)
import functools

import jax
import jax.numpy as jnp
from jax import lax
from jax.experimental import pallas as pl
from jax.experimental.pallas import tpu as pltpu
from jax.experimental.pallas import tpu_sc as plsc

NC = 2
NS = 16
NW = NC * NS
CHUNK = 128


NBUF = 4


def _gather_body(rows_per_w, idx_hbm, table_hbm, out_hbm, idx_v, rows_v, *sems):
    gsems, wsems = sems[:NBUF], sems[NBUF:]
    assert (rows_per_w // CHUNK) % NBUF == 0
    wid = lax.axis_index("s") * NC + lax.axis_index("c")
    base = wid * rows_per_w
    pltpu.sync_copy(idx_hbm.at[pl.ds(base, rows_per_w)], idx_v)
    nchunk = rows_per_w // CHUNK
    ngroups = nchunk // NBUF

    def gather_copy(c, b):
        return pltpu.make_async_copy(
            table_hbm.at[idx_v.at[pl.ds(c * CHUNK, CHUNK)]], rows_v.at[b],
            gsems[b],
        )

    def write_copy(c, b):
        return pltpu.make_async_copy(
            rows_v.at[b], out_hbm.at[pl.ds(base + c * CHUNK, CHUNK)], wsems[b]
        )

    for b in range(NBUF):
        gather_copy(b, b).start()

    def group(g, carry):
        for b in range(NBUF):
            c = g * NBUF + b
            gather_copy(c, b).wait()
            write_copy(c, b).start()
        for b in range(NBUF):
            c = g * NBUF + b
            write_copy(c, b).wait()

            @pl.when(g < ngroups - 1)
            def _():
                gather_copy(c + NBUF, b).start()

        return carry

    lax.fori_loop(0, ngroups, group, 0)


def _sc_gather(idx_flat, table):
    n = idx_flat.shape[0]
    d = table.shape[1]
    rows_per_w = n // NW
    mesh = plsc.VectorSubcoreMesh(core_axis_name="c", subcore_axis_name="s")
    kfn = functools.partial(
        pl.kernel,
        mesh=mesh,
        out_type=jax.ShapeDtypeStruct((n, d), jnp.float32),
        scratch_types=[
            pltpu.VMEM((rows_per_w,), jnp.int32),
            pltpu.VMEM((NBUF, CHUNK, d), jnp.float32),
        ]
        + [pltpu.SemaphoreType.DMA] * (2 * NBUF),
    )(functools.partial(_gather_body, rows_per_w))
    return kfn(idx_flat, table)


def _mlp_body(x_ref, w1_ref, b1_ref, w2_ref, b2_ref, o_ref):
    x = x_ref[...]
    h = lax.dot_general(
        x, w1_ref[...], (((1,), (0,)), ((), ())),
        preferred_element_type=jnp.float32,
    )
    h = jnp.maximum(h + b1_ref[...], 0.0)
    o = lax.dot_general(
        h, w2_ref[...], (((1,), (0,)), ((), ())),
        preferred_element_type=jnp.float32,
    )
    o_ref[...] = jnp.maximum(o + b2_ref[...], 0.0)


def _mlp_alias_body(x_ref, w1_ref, b1_ref, w2_ref, b2_ref, alias_ref, o_ref):
    del alias_ref
    _mlp_body(x_ref, w1_ref, b1_ref, w2_ref, b2_ref, o_ref)


CHUNK_16THS = (4, 4, 4, 4)


def _tc_mlp_chunk(emb, W1, b1, W2, b2, n, blk_off, full):
    c, d = emb.shape
    f = W1.shape[1]
    blk = 8192
    in_specs = [
        pl.BlockSpec((blk, d), lambda i: (i, 0)),
        pl.BlockSpec((d, f), lambda i: (0, 0)),
        pl.BlockSpec((1, f), lambda i: (0, 0)),
        pl.BlockSpec((f, d), lambda i: (0, 0)),
        pl.BlockSpec((1, d), lambda i: (0, 0)),
    ]
    args = [emb, W1, b1[None, :], W2, b2[None, :]]
    kwargs = {}
    body = _mlp_body
    if full is not None:
        body = _mlp_alias_body
        in_specs.append(pl.BlockSpec(memory_space=pl.ANY))
        args.append(full)
        kwargs["input_output_aliases"] = {5: 0}
    return pl.pallas_call(
        body,
        grid=(c // blk,),
        in_specs=in_specs,
        out_specs=pl.BlockSpec((blk, d), lambda i: (i + blk_off, 0)),
        out_shape=jax.ShapeDtypeStruct((n, d), jnp.float32),
        **kwargs,
    )(*args)


def kernel(features, table, W1, b1, W2, b2):
    b, l = features.shape
    d = table.shape[1]
    n = b * l
    idx_flat = features.T.reshape(n).astype(jnp.int32)
    unit = n // 16
    sizes = [e * unit for e in CHUNK_16THS]
    offs = [sum(sizes[:k]) for k in range(len(sizes))]
    full = None
    embs = [
        _sc_gather(lax.dynamic_slice_in_dim(idx_flat, off, sz), table)
        for off, sz in zip(offs, sizes)
    ]
    for emb, off in zip(embs, offs):
        full = _tc_mlp_chunk(emb, W1, b1, W2, b2, n, off // 8192, full)
    return full.reshape(l, b, d).transpose(1, 0, 2)


# speedup vs baseline: 1.0411x; 1.0411x over previous
import functools

import jax
import jax.numpy as jnp
from jax import lax
from jax.experimental import pallas as pl
from jax.experimental.pallas import tpu as pltpu
from jax.experimental.pallas import tpu_sc as plsc

NC = 2
NS = 16
NW = NC * NS
CHUNK = 128


NBUF = 5


def _gather_body(rows_per_w, idx_hbm, table_hbm, out_hbm, idx_v, rows_v, *sems):
    gsems, wsems = sems[:NBUF], sems[NBUF:]
    assert (rows_per_w // CHUNK) % NBUF == 0
    wid = lax.axis_index("s") * NC + lax.axis_index("c")
    base = wid * rows_per_w
    pltpu.sync_copy(idx_hbm.at[pl.ds(base, rows_per_w)], idx_v)
    nchunk = rows_per_w // CHUNK
    ngroups = nchunk // NBUF

    def gather_copy(c, b):
        return pltpu.make_async_copy(
            table_hbm.at[idx_v.at[pl.ds(c * CHUNK, CHUNK)]], rows_v.at[b],
            gsems[b],
        )

    def write_copy(c, b):
        return pltpu.make_async_copy(
            rows_v.at[b], out_hbm.at[pl.ds(base + c * CHUNK, CHUNK)], wsems[b]
        )

    for b in range(NBUF):
        gather_copy(b, b).start()

    def group(g, carry):
        for b in range(NBUF):
            c = g * NBUF + b
            gather_copy(c, b).wait()
            write_copy(c, b).start()
        for b in range(NBUF):
            c = g * NBUF + b
            write_copy(c, b).wait()

            @pl.when(g < ngroups - 1)
            def _():
                gather_copy(c + NBUF, b).start()

        return carry

    lax.fori_loop(0, ngroups, group, 0)


def _sc_gather(idx_flat, table):
    n = idx_flat.shape[0]
    d = table.shape[1]
    rows_per_w = n // NW
    mesh = plsc.VectorSubcoreMesh(core_axis_name="c", subcore_axis_name="s")
    kfn = functools.partial(
        pl.kernel,
        mesh=mesh,
        out_type=jax.ShapeDtypeStruct((n, d), jnp.float32),
        scratch_types=[
            pltpu.VMEM((rows_per_w,), jnp.int32),
            pltpu.VMEM((NBUF, CHUNK, d), jnp.float32),
        ]
        + [pltpu.SemaphoreType.DMA] * (2 * NBUF),
    )(functools.partial(_gather_body, rows_per_w))
    return kfn(idx_flat, table)


def _mlp_body(x_ref, w1_ref, b1_ref, w2_ref, b2_ref, o_ref):
    x = x_ref[...]
    h = lax.dot_general(
        x, w1_ref[...], (((1,), (0,)), ((), ())),
        preferred_element_type=jnp.float32,
    )
    h = jnp.maximum(h + b1_ref[...], 0.0)
    o = lax.dot_general(
        h, w2_ref[...], (((1,), (0,)), ((), ())),
        preferred_element_type=jnp.float32,
    )
    o_ref[...] = jnp.maximum(o + b2_ref[...], 0.0)


def _mlp_alias_body(x_ref, w1_ref, b1_ref, w2_ref, b2_ref, alias_ref, o_ref):
    del alias_ref
    _mlp_body(x_ref, w1_ref, b1_ref, w2_ref, b2_ref, o_ref)


CHUNK_16THS = (4, 4, 4, 4)


def _tc_mlp_chunk(emb, W1, b1, W2, b2, n, blk_off, full):
    c, d = emb.shape
    f = W1.shape[1]
    blk = 16384
    in_specs = [
        pl.BlockSpec((blk, d), lambda i: (i, 0)),
        pl.BlockSpec((d, f), lambda i: (0, 0)),
        pl.BlockSpec((1, f), lambda i: (0, 0)),
        pl.BlockSpec((f, d), lambda i: (0, 0)),
        pl.BlockSpec((1, d), lambda i: (0, 0)),
    ]
    args = [emb, W1, b1[None, :], W2, b2[None, :]]
    kwargs = {}
    body = _mlp_body
    if full is not None:
        body = _mlp_alias_body
        in_specs.append(pl.BlockSpec(memory_space=pl.ANY))
        args.append(full)
        kwargs["input_output_aliases"] = {5: 0}
    return pl.pallas_call(
        body,
        grid=(c // blk,),
        in_specs=in_specs,
        out_specs=pl.BlockSpec((blk, d), lambda i: (i + blk_off, 0)),
        out_shape=jax.ShapeDtypeStruct((n, d), jnp.float32),
        **kwargs,
    )(*args)


def kernel(features, table, W1, b1, W2, b2):
    b, l = features.shape
    d = table.shape[1]
    n = b * l
    idx_flat = features.T.reshape(n).astype(jnp.int32)
    unit = n // 16
    sizes = [e * unit for e in CHUNK_16THS]
    offs = [sum(sizes[:k]) for k in range(len(sizes))]
    full = None
    embs = [
        _sc_gather(lax.dynamic_slice_in_dim(idx_flat, off, sz), table)
        for off, sz in zip(offs, sizes)
    ]
    for emb, off in zip(embs, offs):
        full = _tc_mlp_chunk(emb, W1, b1, W2, b2, n, off // 16384, full)
    return full.reshape(l, b, d).transpose(1, 0, 2)


# speedup vs baseline: 1.0524x; 1.0108x over previous
import functools

import jax
import jax.numpy as jnp
from jax import lax
from jax.experimental import pallas as pl
from jax.experimental.pallas import tpu as pltpu
from jax.experimental.pallas import tpu_sc as plsc

NC = 2
NS = 16
NW = NC * NS
CHUNK = 128


NBUF = 5


def _gather_body(rows_per_w, idx_hbm, table_hbm, out_hbm, idx_v, rows_v, *sems):
    gsems, wsems = sems[:NBUF], sems[NBUF:]
    assert (rows_per_w // CHUNK) % NBUF == 0
    wid = lax.axis_index("s") * NC + lax.axis_index("c")
    base = wid * rows_per_w
    pltpu.sync_copy(idx_hbm.at[pl.ds(base, rows_per_w)], idx_v)
    nchunk = rows_per_w // CHUNK
    ngroups = nchunk // NBUF

    def gather_copy(c, b):
        return pltpu.make_async_copy(
            table_hbm.at[idx_v.at[pl.ds(c * CHUNK, CHUNK)]], rows_v.at[b],
            gsems[b],
        )

    def write_copy(c, b):
        return pltpu.make_async_copy(
            rows_v.at[b], out_hbm.at[pl.ds(base + c * CHUNK, CHUNK)], wsems[b]
        )

    for b in range(NBUF):
        gather_copy(b, b).start()

    def group(g, carry):
        for b in range(NBUF):
            c = g * NBUF + b
            gather_copy(c, b).wait()
            write_copy(c, b).start()
        for b in range(NBUF):
            c = g * NBUF + b
            write_copy(c, b).wait()

            @pl.when(g < ngroups - 1)
            def _():
                gather_copy(c + NBUF, b).start()

        return carry

    lax.fori_loop(0, ngroups, group, 0)


def _sc_gather(idx_flat, table):
    n = idx_flat.shape[0]
    d = table.shape[1]
    rows_per_w = n // NW
    mesh = plsc.VectorSubcoreMesh(core_axis_name="c", subcore_axis_name="s")
    kfn = functools.partial(
        pl.kernel,
        mesh=mesh,
        out_type=jax.ShapeDtypeStruct((n, d), jnp.float32),
        scratch_types=[
            pltpu.VMEM((rows_per_w,), jnp.int32),
            pltpu.VMEM((NBUF, CHUNK, d), jnp.float32),
        ]
        + [pltpu.SemaphoreType.DMA] * (2 * NBUF),
    )(functools.partial(_gather_body, rows_per_w))
    return kfn(idx_flat, table)


def _mlp_body(x_ref, w1_ref, b1_ref, w2_ref, b2_ref, o_ref):
    x = x_ref[...]
    h = lax.dot_general(
        x, w1_ref[...], (((1,), (0,)), ((), ())),
        preferred_element_type=jnp.float32,
    )
    h = jnp.maximum(h + b1_ref[...], 0.0)
    o = lax.dot_general(
        h, w2_ref[...], (((1,), (0,)), ((), ())),
        preferred_element_type=jnp.float32,
    )
    o_ref[...] = jnp.maximum(o + b2_ref[...], 0.0)


def _mlp_alias_body(x_ref, w1_ref, b1_ref, w2_ref, b2_ref, alias_ref, o_ref):
    del alias_ref
    _mlp_body(x_ref, w1_ref, b1_ref, w2_ref, b2_ref, o_ref)


CHUNK_16THS = (4, 4, 4, 4)


def _tc_mlp_chunk(emb, W1, b1, W2, b2, n, blk_off, full):
    c, d = emb.shape
    f = W1.shape[1]
    blk = 20480
    in_specs = [
        pl.BlockSpec((blk, d), lambda i: (i, 0)),
        pl.BlockSpec((d, f), lambda i: (0, 0)),
        pl.BlockSpec((1, f), lambda i: (0, 0)),
        pl.BlockSpec((f, d), lambda i: (0, 0)),
        pl.BlockSpec((1, d), lambda i: (0, 0)),
    ]
    args = [emb, W1, b1[None, :], W2, b2[None, :]]
    kwargs = {}
    body = _mlp_body
    if full is not None:
        body = _mlp_alias_body
        in_specs.append(pl.BlockSpec(memory_space=pl.ANY))
        args.append(full)
        kwargs["input_output_aliases"] = {5: 0}
    return pl.pallas_call(
        body,
        grid=(c // blk,),
        in_specs=in_specs,
        out_specs=pl.BlockSpec((blk, d), lambda i: (i + blk_off, 0)),
        out_shape=jax.ShapeDtypeStruct((n, d), jnp.float32),
        **kwargs,
    )(*args)


def kernel(features, table, W1, b1, W2, b2):
    b, l = features.shape
    d = table.shape[1]
    n = b * l
    idx_flat = features.T.reshape(n).astype(jnp.int32)
    unit = n // 16
    sizes = [e * unit for e in CHUNK_16THS]
    offs = [sum(sizes[:k]) for k in range(len(sizes))]
    full = None
    embs = [
        _sc_gather(lax.dynamic_slice_in_dim(idx_flat, off, sz), table)
        for off, sz in zip(offs, sizes)
    ]
    for emb, off in zip(embs, offs):
        full = _tc_mlp_chunk(emb, W1, b1, W2, b2, n, off // 20480, full)
    return full.reshape(l, b, d).transpose(1, 0, 2)
